# baseline (speedup 1.0000x reference)
import functools

import jax
import jax.numpy as jnp
from jax.experimental import pallas as pl
from jax.experimental.pallas import tpu as pltpu
from jax.experimental.pallas import tpu_sc as plsc

B = 8
C = 32
T = 48 * 48
K = 9
KPAD = 16

R = 256
NT = T // R

NC = 2
NS = 16
NW = NC * NS
NTOT = B * T * K
PER_W = NTOT // NW
CH = 96
NCH = PER_W // CH


def _topk_body(xf_ref, xr_ref, idx_ref, xt_ref, sim_ref, m_ref):
    b = pl.program_id(0)
    xfb = xf_ref[0]
    norm = jnp.sqrt(jnp.sum(xfb * xfb, axis=0, keepdims=True))
    xn = xfb / jnp.maximum(norm, 1e-12)
    rows = xr_ref[0]
    norm_r = jnp.sqrt(jnp.sum(rows * rows, axis=0, keepdims=True))
    rows_n = rows / jnp.maximum(norm_r, 1e-12)
    xt_ref[0] = rows.T

    sim = jax.lax.dot_general(
        xn, rows_n, (((0,), (0,)), ((), ())),
        preferred_element_type=jnp.float32)
    sim = jnp.clip(sim, -1.0, 1.0)

    iota_i = jax.lax.broadcasted_iota(jnp.int32, (3, T), 1)
    row_id = jax.lax.broadcasted_iota(jnp.int32, (3, T), 0)
    lhs3 = jnp.where(
        row_id == 0, iota_i // 256,
        jnp.where(row_id == 1, iota_i % 256, 1)).astype(jnp.bfloat16)

    sim_ref[...] = sim
    m_ref[0:1] = jnp.max(sim, axis=0, keepdims=True)

    for k in range(K):
        v = sim_ref[...]
        m = m_ref[0:1]
        ge = v >= m
        gef = jnp.where(ge, 1.0, 0.0).astype(jnp.bfloat16)
        ms = jax.lax.dot_general(
            lhs3, gef, (((1,), (0,)), ((), ())),
            preferred_element_type=jnp.float32)
        cnt = ms[2:3]
        tied = jnp.max(cnt) > 1.5

        @pl.when(jnp.logical_not(tied))
        def _():
            ikf = (ms[0:1] * 256.0 + ms[1:2]) / cnt
            ik = jnp.clip((ikf + 0.5).astype(jnp.int32), 0, T - 1)
            idx_ref[0, pl.ds(k, 1), :] = ik + b * T
            if k < K - 1:
                nv = jnp.where(ge, -3.0, v)
                sim_ref[...] = nv
                m_ref[0:1] = jnp.max(nv, axis=0, keepdims=True)

        @pl.when(tied)
        def _():
            iota = jax.lax.broadcasted_iota(jnp.int32, (T, R), 0)
            cand = jnp.where(ge, iota, T)
            ik = jnp.min(cand, axis=0, keepdims=True)
            idx_ref[0, pl.ds(k, 1), :] = ik + b * T
            if k < K - 1:
                nv = jnp.where(iota == ik, -3.0, v)
                sim_ref[...] = nv
                m_ref[0:1] = jnp.max(nv, axis=0, keepdims=True)

    idx_ref[0, pl.ds(K, KPAD - K), :] = jnp.full((KPAD - K, R), b * T,
                                                 jnp.int32)


_topk_call = pl.pallas_call(
    _topk_body,
    grid=(B, NT),
    in_specs=[pl.BlockSpec((1, C, T), lambda b, rt: (b, 0, 0)),
              pl.BlockSpec((1, C, R), lambda b, rt: (b, 0, rt))],
    out_specs=[
        pl.BlockSpec((1, KPAD, R), lambda b, rt: (b, 0, rt)),
        pl.BlockSpec((1, R, C), lambda b, rt: (b, rt, 0)),
    ],
    out_shape=[
        jax.ShapeDtypeStruct((B, KPAD, T), jnp.int32),
        jax.ShapeDtypeStruct((B, T, C), jnp.float32),
    ],
    scratch_shapes=[
        pltpu.VMEM((T, R), jnp.float32),
        pltpu.VMEM((8, R), jnp.float32),
    ],
)


def _gather_body(tab_ref, idx_ref, out_ref, idx_v, rows_v, sem):
    c = jax.lax.axis_index("c")
    s = jax.lax.axis_index("s")
    wid = s * NC + c
    pltpu.sync_copy(idx_ref.at[wid], idx_v)

    def chunk(j, carry):
        pltpu.async_copy(tab_ref.at[idx_v.at[j]], rows_v, sem).wait()
        pltpu.sync_copy(rows_v, out_ref.at[pl.ds(wid * PER_W + j * CH, CH)])
        return carry

    jax.lax.fori_loop(0, NCH, chunk, 0)


@functools.cache
def _make_gather_call():
    return pl.kernel(
        _gather_body,
        out_type=jax.ShapeDtypeStruct((NTOT, C), jnp.float32),
        mesh=plsc.VectorSubcoreMesh(core_axis_name="c", subcore_axis_name="s",
                                    num_cores=NC, num_subcores=NS),
        scratch_types=[
            pltpu.VMEM((NCH, CH), jnp.int32),
            pltpu.VMEM((CH, C), jnp.float32),
            pltpu.SemaphoreType.DMA,
        ],
        compiler_params=pltpu.CompilerParams(use_tc_tiling_on_sc=False),
    )


def _conv_body(p_ref, w_ref, b_ref, o_ref):
    k = pl.program_id(1)
    contrib = jax.lax.dot_general(
        w_ref[0], p_ref[0], (((1,), (1,)), ((), ())),
        preferred_element_type=jnp.float32)

    @pl.when(k == 0)
    def _():
        o_ref[0] = contrib + b_ref[...]

    @pl.when(k != 0)
    def _():
        o_ref[0] = o_ref[0] + contrib


_conv_call = pl.pallas_call(
    _conv_body,
    grid=(B, K),
    in_specs=[
        pl.BlockSpec((1, T, C), lambda b, k: (b * K + k, 0, 0)),
        pl.BlockSpec((1, C, C), lambda b, k: (k, 0, 0)),
        pl.BlockSpec((C, 1), lambda b, k: (0, 0)),
    ],
    out_specs=pl.BlockSpec((1, C, T), lambda b, k: (b, 0, 0)),
    out_shape=jax.ShapeDtypeStruct((B, C, T), jnp.float32),
)


def kernel(x, W, b):
    xf = x.reshape(B, C, T)
    idx_g, xt = _topk_call(xf, xf)
    idx3 = idx_g[:, :K, :].reshape(NW, NCH, CH)
    prime = _make_gather_call()(xt.reshape(B * T, C), idx3)
    p3 = prime.reshape(B * K, T, C)
    w9 = W.transpose(2, 0, 1)
    out = _conv_call(p3, w9, b.reshape(C, 1))
    return out.reshape(B, C, 48, 48)


# speedup vs baseline: 1.4185x; 1.4185x over previous
import functools

import jax
import jax.numpy as jnp
from jax.experimental import pallas as pl
from jax.experimental.pallas import tpu as pltpu
from jax.experimental.pallas import tpu_sc as plsc

B = 8
C = 32
T = 48 * 48
K = 9
KPAD = 16

R = 256
NT = T // R

NC = 2
NS = 16
NW = NC * NS
NTOT = B * T * K
PER_W = NTOT // NW
CH = 96
NCH = PER_W // CH


def _topk_body(xf_ref, xr_ref, idx_ref, xt_ref):
    b = pl.program_id(0)
    xfb = xf_ref[0]
    norm = jnp.sqrt(jnp.sum(xfb * xfb, axis=0, keepdims=True))
    xn = xfb / jnp.maximum(norm, 1e-12)
    rows = xr_ref[0]
    norm_r = jnp.sqrt(jnp.sum(rows * rows, axis=0, keepdims=True))
    rows_n = rows / jnp.maximum(norm_r, 1e-12)
    xt_ref[0] = rows.T

    sim = jax.lax.dot_general(
        xn, rows_n, (((0,), (0,)), ((), ())),
        preferred_element_type=jnp.float32)
    sim = jnp.clip(sim, -1.0, 1.0)

    iota_i = jax.lax.broadcasted_iota(jnp.int32, (3, T), 1)
    row_id = jax.lax.broadcasted_iota(jnp.int32, (3, T), 0)
    lhs3 = jnp.where(
        row_id == 0, iota_i // 256,
        jnp.where(row_id == 1, iota_i % 256, 1)).astype(jnp.bfloat16)

    sim0 = sim
    m = jnp.max(sim, axis=0, keepdims=True)
    picks = []
    tied = False
    for k in range(K):
        ge = sim >= m
        gef = jnp.where(ge, 1.0, 0.0).astype(jnp.bfloat16)
        ms = jax.lax.dot_general(
            lhs3, gef, (((1,), (0,)), ((), ())),
            preferred_element_type=jnp.float32)
        tied = jnp.logical_or(tied, jnp.max(ms[2:3]) > 1.5)
        ikf = (ms[0:1] * 256.0 + ms[1:2]) / ms[2:3]
        ik = jnp.clip((ikf + 0.5).astype(jnp.int32), 0, T - 1)
        picks.append(ik)
        if k < K - 1:
            sim = jnp.where(ge, -3.0, sim)
            m = jnp.max(sim, axis=0, keepdims=True)
    idx = jnp.concatenate(picks + [jnp.zeros((KPAD - K, R), jnp.int32)], axis=0)
    idx_ref[0] = idx + b * T

    @pl.when(tied)
    def _():
        iota = jax.lax.broadcasted_iota(jnp.int32, (T, R), 0)
        s2 = sim0
        for k in range(K):
            m2 = jnp.max(s2, axis=0, keepdims=True)
            cand = jnp.where(s2 >= m2, iota, T)
            ik2 = jnp.min(cand, axis=0, keepdims=True)
            idx_ref[0, pl.ds(k, 1), :] = ik2 + b * T
            if k < K - 1:
                s2 = jnp.where(iota == ik2, -3.0, s2)


_topk_call = pl.pallas_call(
    _topk_body,
    grid=(B, NT),
    in_specs=[pl.BlockSpec((1, C, T), lambda b, rt: (b, 0, 0)),
              pl.BlockSpec((1, C, R), lambda b, rt: (b, 0, rt))],
    out_specs=[
        pl.BlockSpec((1, KPAD, R), lambda b, rt: (b, 0, rt)),
        pl.BlockSpec((1, R, C), lambda b, rt: (b, rt, 0)),
    ],
    out_shape=[
        jax.ShapeDtypeStruct((B, KPAD, T), jnp.int32),
        jax.ShapeDtypeStruct((B, T, C), jnp.float32),
    ],
)


def _gather_body(tab_ref, idx_ref, out_ref, idx_v, rows_v, sem):
    c = jax.lax.axis_index("c")
    s = jax.lax.axis_index("s")
    wid = s * NC + c
    pltpu.sync_copy(idx_ref.at[wid], idx_v)

    def chunk(j, carry):
        pltpu.async_copy(tab_ref.at[idx_v.at[j]], rows_v, sem).wait()
        pltpu.sync_copy(rows_v, out_ref.at[pl.ds(wid * PER_W + j * CH, CH)])
        return carry

    jax.lax.fori_loop(0, NCH, chunk, 0)


@functools.cache
def _make_gather_call():
    return pl.kernel(
        _gather_body,
        out_type=jax.ShapeDtypeStruct((NTOT, C), jnp.float32),
        mesh=plsc.VectorSubcoreMesh(core_axis_name="c", subcore_axis_name="s",
                                    num_cores=NC, num_subcores=NS),
        scratch_types=[
            pltpu.VMEM((NCH, CH), jnp.int32),
            pltpu.VMEM((CH, C), jnp.float32),
            pltpu.SemaphoreType.DMA,
        ],
        compiler_params=pltpu.CompilerParams(use_tc_tiling_on_sc=False),
    )


def _conv_body(p_ref, w_ref, b_ref, o_ref):
    k = pl.program_id(1)
    contrib = jax.lax.dot_general(
        w_ref[0], p_ref[0], (((1,), (1,)), ((), ())),
        preferred_element_type=jnp.float32)

    @pl.when(k == 0)
    def _():
        o_ref[0] = contrib + b_ref[...]

    @pl.when(k != 0)
    def _():
        o_ref[0] = o_ref[0] + contrib


_conv_call = pl.pallas_call(
    _conv_body,
    grid=(B, K),
    in_specs=[
        pl.BlockSpec((1, T, C), lambda b, k: (b * K + k, 0, 0)),
        pl.BlockSpec((1, C, C), lambda b, k: (k, 0, 0)),
        pl.BlockSpec((C, 1), lambda b, k: (0, 0)),
    ],
    out_specs=pl.BlockSpec((1, C, T), lambda b, k: (b, 0, 0)),
    out_shape=jax.ShapeDtypeStruct((B, C, T), jnp.float32),
)


def kernel(x, W, b):
    xf = x.reshape(B, C, T)
    idx_g, xt = _topk_call(xf, xf)
    idx3 = idx_g[:, :K, :].reshape(NW, NCH, CH)
    prime = _make_gather_call()(xt.reshape(B * T, C), idx3)
    p3 = prime.reshape(B * K, T, C)
    w9 = W.transpose(2, 0, 1)
    out = _conv_call(p3, w9, b.reshape(C, 1))
    return out.reshape(B, C, 48, 48)


# speedup vs baseline: 1.5198x; 1.0714x over previous
import functools

import jax
import jax.numpy as jnp
from jax.experimental import pallas as pl
from jax.experimental.pallas import tpu as pltpu
from jax.experimental.pallas import tpu_sc as plsc

B = 8
C = 32
T = 48 * 48
K = 9
KPAD = 16

R = 256
NT = T // R

NC = 2
NS = 16
NW = NC * NS
CH = 96


def _topk_body(xf_ref, xr_ref, idx_ref, xt_ref):
    b = pl.program_id(0)
    xfb = xf_ref[0]
    norm = jnp.sqrt(jnp.sum(xfb * xfb, axis=0, keepdims=True))
    xn = xfb / jnp.maximum(norm, 1e-12)
    rows = xr_ref[0]
    norm_r = jnp.sqrt(jnp.sum(rows * rows, axis=0, keepdims=True))
    rows_n = rows / jnp.maximum(norm_r, 1e-12)
    xt_ref[0] = rows.T

    sim = jax.lax.dot_general(
        xn, rows_n, (((0,), (0,)), ((), ())),
        preferred_element_type=jnp.float32)
    sim = jnp.clip(sim, -1.0, 1.0)

    iota_i = jax.lax.broadcasted_iota(jnp.int32, (3, T), 1)
    row_id = jax.lax.broadcasted_iota(jnp.int32, (3, T), 0)
    lhs3 = jnp.where(
        row_id == 0, iota_i // 256,
        jnp.where(row_id == 1, iota_i % 256, 1)).astype(jnp.bfloat16)

    sim0 = sim
    m = jnp.max(sim, axis=0, keepdims=True)
    picks = []
    tied = False
    for k in range(K):
        ge = sim >= m
        gef = jnp.where(ge, 1.0, 0.0).astype(jnp.bfloat16)
        ms = jax.lax.dot_general(
            lhs3, gef, (((1,), (0,)), ((), ())),
            preferred_element_type=jnp.float32)
        tied = jnp.logical_or(tied, jnp.max(ms[2:3]) > 1.5)
        ikf = (ms[0:1] * 256.0 + ms[1:2]) / ms[2:3]
        ik = jnp.clip((ikf + 0.5).astype(jnp.int32), 0, T - 1)
        picks.append(ik)
        if k < K - 1:
            sim = jnp.where(ge, -3.0, sim)
            m = jnp.max(sim, axis=0, keepdims=True)
    idx = jnp.concatenate(picks + [jnp.zeros((KPAD - K, R), jnp.int32)], axis=0)
    idx_ref[0] = idx + b * T

    @pl.when(tied)
    def _():
        iota = jax.lax.broadcasted_iota(jnp.int32, (T, R), 0)
        s2 = sim0
        for k in range(K):
            m2 = jnp.max(s2, axis=0, keepdims=True)
            cand = jnp.where(s2 >= m2, iota, T)
            ik2 = jnp.min(cand, axis=0, keepdims=True)
            idx_ref[0, pl.ds(k, 1), :] = ik2 + b * T
            if k < K - 1:
                s2 = jnp.where(iota == ik2, -3.0, s2)


@functools.cache
def _make_topk_call(bc):
    return pl.pallas_call(
        _topk_body,
        grid=(bc, NT),
        in_specs=[pl.BlockSpec((1, C, T), lambda b, rt: (b, 0, 0)),
                  pl.BlockSpec((1, C, R), lambda b, rt: (b, 0, rt))],
        out_specs=[
            pl.BlockSpec((1, KPAD, R), lambda b, rt: (b, 0, rt)),
            pl.BlockSpec((1, R, C), lambda b, rt: (b, rt, 0)),
        ],
        out_shape=[
            jax.ShapeDtypeStruct((bc, KPAD, T), jnp.int32),
            jax.ShapeDtypeStruct((bc, T, C), jnp.float32),
        ],
    )


def _gather_body(nch, tab_ref, idx_ref, out_ref, idx_v, rows_v, sem):
    per_w = nch * CH
    c = jax.lax.axis_index("c")
    s = jax.lax.axis_index("s")
    wid = s * NC + c
    pltpu.sync_copy(idx_ref.at[wid], idx_v)

    def chunk(j, carry):
        pltpu.async_copy(tab_ref.at[idx_v.at[j]], rows_v, sem).wait()
        pltpu.sync_copy(rows_v, out_ref.at[pl.ds(wid * per_w + j * CH, CH)])
        return carry

    jax.lax.fori_loop(0, nch, chunk, 0)


@functools.cache
def _make_gather_call(bc):
    ntot = bc * T * K
    nch = ntot // (NW * CH)
    return pl.kernel(
        functools.partial(_gather_body, nch),
        out_type=jax.ShapeDtypeStruct((ntot, C), jnp.float32),
        mesh=plsc.VectorSubcoreMesh(core_axis_name="c", subcore_axis_name="s",
                                    num_cores=NC, num_subcores=NS),
        scratch_types=[
            pltpu.VMEM((nch, CH), jnp.int32),
            pltpu.VMEM((CH, C), jnp.float32),
            pltpu.SemaphoreType.DMA,
        ],
        compiler_params=pltpu.CompilerParams(use_tc_tiling_on_sc=False),
    )


def _conv_body(p_ref, w_ref, b_ref, o_ref):
    k = pl.program_id(1)
    contrib = jax.lax.dot_general(
        w_ref[0], p_ref[0], (((1,), (1,)), ((), ())),
        preferred_element_type=jnp.float32)

    @pl.when(k == 0)
    def _():
        o_ref[0] = contrib + b_ref[...]

    @pl.when(k != 0)
    def _():
        o_ref[0] = o_ref[0] + contrib


@functools.cache
def _make_conv_call(bc):
    return pl.pallas_call(
        _conv_body,
        grid=(bc, K),
        in_specs=[
            pl.BlockSpec((1, T, C), lambda b, k: (b * K + k, 0, 0)),
            pl.BlockSpec((1, C, C), lambda b, k: (k, 0, 0)),
            pl.BlockSpec((C, 1), lambda b, k: (0, 0)),
        ],
        out_specs=pl.BlockSpec((1, C, T), lambda b, k: (b, 0, 0)),
        out_shape=jax.ShapeDtypeStruct((bc, C, T), jnp.float32),
    )


def _chunk_pipeline(xf, w9, bias, bc):
    ntot = bc * T * K
    nch = ntot // (NW * CH)
    idx_g, xt = _make_topk_call(bc)(xf, xf)
    idx3 = idx_g[:, :K, :].reshape(NW, nch, CH)
    prime = _make_gather_call(bc)(xt.reshape(bc * T, C), idx3)
    p3 = prime.reshape(bc * K, T, C)
    return _make_conv_call(bc)(p3, w9, bias)


def kernel(x, W, b):
    xf = x.reshape(B, C, T)
    w9 = W.transpose(2, 0, 1)
    bias = b.reshape(C, 1)
    half = B // 2
    out0 = _chunk_pipeline(xf[:half], w9, bias, half)
    out1 = _chunk_pipeline(xf[half:], w9, bias, half)
    return jnp.concatenate([out0, out1], axis=0).reshape(B, C, 48, 48)


# speedup vs baseline: 1.5245x; 1.0031x over previous
import functools

import jax
import jax.numpy as jnp
from jax.experimental import pallas as pl
from jax.experimental.pallas import tpu as pltpu
from jax.experimental.pallas import tpu_sc as plsc

B = 8
C = 32
T = 48 * 48
K = 9
KPAD = 16

R = 256
NT = T // R

NC = 2
NS = 16
NW = NC * NS
CH = 96


def _topk_body(xf_ref, xr_ref, idx_ref, xt_ref):
    b = pl.program_id(0)
    xfb = xf_ref[0]
    norm = jnp.sqrt(jnp.sum(xfb * xfb, axis=0, keepdims=True))
    xn = xfb / jnp.maximum(norm, 1e-12)
    rows = xr_ref[0]
    norm_r = jnp.sqrt(jnp.sum(rows * rows, axis=0, keepdims=True))
    rows_n = rows / jnp.maximum(norm_r, 1e-12)
    xt_ref[0] = rows.T

    sim = jax.lax.dot_general(
        xn, rows_n, (((0,), (0,)), ((), ())),
        preferred_element_type=jnp.float32)
    sim = jnp.clip(sim, -1.0, 1.0)

    iota_i = jax.lax.broadcasted_iota(jnp.int32, (3, T), 1)
    row_id = jax.lax.broadcasted_iota(jnp.int32, (3, T), 0)
    lhs3 = jnp.where(
        row_id == 0, iota_i // 256,
        jnp.where(row_id == 1, iota_i % 256, 1)).astype(jnp.bfloat16)

    sim0 = sim
    m = jnp.max(sim, axis=0, keepdims=True)
    picks = []
    tied = False
    for k in range(K):
        ge = sim >= m
        gef = jnp.where(ge, 1.0, 0.0).astype(jnp.bfloat16)
        ms = jax.lax.dot_general(
            lhs3, gef, (((1,), (0,)), ((), ())),
            preferred_element_type=jnp.float32)
        tied = jnp.logical_or(tied, jnp.max(ms[2:3]) > 1.5)
        ikf = (ms[0:1] * 256.0 + ms[1:2]) / ms[2:3]
        ik = jnp.clip((ikf + 0.5).astype(jnp.int32), 0, T - 1)
        picks.append(ik)
        if k < K - 1:
            sim = jnp.where(ge, -3.0, sim)
            m = jnp.max(sim, axis=0, keepdims=True)
    idx = jnp.concatenate(picks + [jnp.zeros((KPAD - K, R), jnp.int32)], axis=0)
    idx_ref[0] = idx + b * T

    @pl.when(tied)
    def _():
        iota = jax.lax.broadcasted_iota(jnp.int32, (T, R), 0)
        s2 = sim0
        for k in range(K):
            m2 = jnp.max(s2, axis=0, keepdims=True)
            cand = jnp.where(s2 >= m2, iota, T)
            ik2 = jnp.min(cand, axis=0, keepdims=True)
            idx_ref[0, pl.ds(k, 1), :] = ik2 + b * T
            if k < K - 1:
                s2 = jnp.where(iota == ik2, -3.0, s2)


@functools.cache
def _make_topk_call(bc):
    return pl.pallas_call(
        _topk_body,
        grid=(bc, NT),
        in_specs=[pl.BlockSpec((1, C, T), lambda b, rt: (b, 0, 0)),
                  pl.BlockSpec((1, C, R), lambda b, rt: (b, 0, rt))],
        out_specs=[
            pl.BlockSpec((1, KPAD, R), lambda b, rt: (b, 0, rt)),
            pl.BlockSpec((1, R, C), lambda b, rt: (b, rt, 0)),
        ],
        out_shape=[
            jax.ShapeDtypeStruct((bc, KPAD, T), jnp.int32),
            jax.ShapeDtypeStruct((bc, T, C), jnp.float32),
        ],
    )


NBUF = 3


def _gather_body(nch, tab_ref, idx_ref, out_ref, idx_v,
                 rows0, rows1, rows2, sem0, sem1, sem2):
    per_w = nch * CH
    c = jax.lax.axis_index("c")
    s = jax.lax.axis_index("s")
    wid = s * NC + c
    base = wid * per_w
    rows = (rows0, rows1, rows2)
    sems = (sem0, sem1, sem2)
    pltpu.sync_copy(idx_ref.at[wid], idx_v)

    for u in range(NBUF):
        pltpu.async_copy(tab_ref.at[idx_v.at[u]], rows[u], sems[u])

    def chunk(i, carry):
        for u in range(NBUF):
            j = NBUF * i + u
            pltpu.make_async_copy(tab_ref.at[idx_v.at[j]],
                                  rows[u], sems[u]).wait()
            pltpu.sync_copy(rows[u], out_ref.at[pl.ds(base + j * CH, CH)])

            @pl.when(j + NBUF < nch)
            def _(u=u, j=j):
                pltpu.async_copy(tab_ref.at[idx_v.at[j + NBUF]],
                                 rows[u], sems[u])
        return carry

    jax.lax.fori_loop(0, nch // NBUF, chunk, 0)


@functools.cache
def _make_gather_call(bc):
    ntot = bc * T * K
    nch = ntot // (NW * CH)
    assert nch % NBUF == 0
    return pl.kernel(
        functools.partial(_gather_body, nch),
        out_type=jax.ShapeDtypeStruct((ntot, C), jnp.float32),
        mesh=plsc.VectorSubcoreMesh(core_axis_name="c", subcore_axis_name="s",
                                    num_cores=NC, num_subcores=NS),
        scratch_types=[
            pltpu.VMEM((nch, CH), jnp.int32),
            pltpu.VMEM((CH, C), jnp.float32),
            pltpu.VMEM((CH, C), jnp.float32),
            pltpu.VMEM((CH, C), jnp.float32),
            pltpu.SemaphoreType.DMA,
            pltpu.SemaphoreType.DMA,
            pltpu.SemaphoreType.DMA,
        ],
        compiler_params=pltpu.CompilerParams(use_tc_tiling_on_sc=False),
    )


def _conv_body(p_ref, w_ref, b_ref, o_ref):
    k = pl.program_id(1)
    contrib = jax.lax.dot_general(
        w_ref[0], p_ref[0], (((1,), (1,)), ((), ())),
        preferred_element_type=jnp.float32)

    @pl.when(k == 0)
    def _():
        o_ref[0] = contrib + b_ref[...]

    @pl.when(k != 0)
    def _():
        o_ref[0] = o_ref[0] + contrib


@functools.cache
def _make_conv_call(bc):
    return pl.pallas_call(
        _conv_body,
        grid=(bc, K),
        in_specs=[
            pl.BlockSpec((1, T, C), lambda b, k: (b * K + k, 0, 0)),
            pl.BlockSpec((1, C, C), lambda b, k: (k, 0, 0)),
            pl.BlockSpec((C, 1), lambda b, k: (0, 0)),
        ],
        out_specs=pl.BlockSpec((1, C, T), lambda b, k: (b, 0, 0)),
        out_shape=jax.ShapeDtypeStruct((bc, C, T), jnp.float32),
    )


def _chunk_pipeline(xf, w9, bias, bc):
    ntot = bc * T * K
    nch = ntot // (NW * CH)
    idx_g, xt = _make_topk_call(bc)(xf, xf)
    idx3 = idx_g[:, :K, :].reshape(NW, nch, CH)
    prime = _make_gather_call(bc)(xt.reshape(bc * T, C), idx3)
    p3 = prime.reshape(bc * K, T, C)
    return _make_conv_call(bc)(p3, w9, bias)


def kernel(x, W, b):
    xf = x.reshape(B, C, T)
    w9 = W.transpose(2, 0, 1)
    bias = b.reshape(C, 1)
    half = B // 2
    out0 = _chunk_pipeline(xf[:half], w9, bias, half)
    out1 = _chunk_pipeline(xf[half:], w9, bias, half)
    return jnp.concatenate([out0, out1], axis=0).reshape(B, C, 48, 48)


# speedup vs baseline: 1.7767x; 1.1655x over previous
import functools

import jax
import jax.numpy as jnp
from jax.experimental import pallas as pl
from jax.experimental.pallas import tpu as pltpu
from jax.experimental.pallas import tpu_sc as plsc

B = 8
C = 32
T = 48 * 48
K = 9
KPAD = 16

R = 384
NT = T // R

NC = 2
NS = 16
NW = NC * NS
CH = 96


def _topk_body(xf_ref, xr_ref, idx_ref, xt_ref):
    b = pl.program_id(0)
    xfb = xf_ref[0]
    norm = jnp.sqrt(jnp.sum(xfb * xfb, axis=0, keepdims=True))
    xn = xfb / jnp.maximum(norm, 1e-12)
    rows = xr_ref[0]
    norm_r = jnp.sqrt(jnp.sum(rows * rows, axis=0, keepdims=True))
    rows_n = rows / jnp.maximum(norm_r, 1e-12)
    xt_ref[0] = rows.T

    sim = jax.lax.dot_general(
        xn, rows_n, (((0,), (0,)), ((), ())),
        preferred_element_type=jnp.float32)
    sim = jnp.clip(sim, -1.0, 1.0)

    iota_i = jax.lax.broadcasted_iota(jnp.int32, (3, T), 1)
    row_id = jax.lax.broadcasted_iota(jnp.int32, (3, T), 0)
    lhs3 = jnp.where(
        row_id == 0, iota_i // 256,
        jnp.where(row_id == 1, iota_i % 256, 1)).astype(jnp.bfloat16)

    sim0 = sim
    m = jnp.max(sim, axis=0, keepdims=True)
    picks = []
    tied = False
    for k in range(K):
        ge = sim >= m
        gef = jnp.where(ge, 1.0, 0.0).astype(jnp.bfloat16)
        ms = jax.lax.dot_general(
            lhs3, gef, (((1,), (0,)), ((), ())),
            preferred_element_type=jnp.float32)
        tied = jnp.logical_or(tied, jnp.max(ms[2:3]) > 1.5)
        ikf = (ms[0:1] * 256.0 + ms[1:2]) / ms[2:3]
        ik = jnp.clip((ikf + 0.5).astype(jnp.int32), 0, T - 1)
        picks.append(ik)
        if k < K - 1:
            sim = jnp.where(ge, -3.0, sim)
            m = jnp.max(sim, axis=0, keepdims=True)
    idx = jnp.concatenate(picks + [jnp.zeros((KPAD - K, R), jnp.int32)], axis=0)
    idx_ref[0] = idx + b * T

    @pl.when(tied)
    def _():
        iota = jax.lax.broadcasted_iota(jnp.int32, (T, R), 0)
        s2 = sim0
        for k in range(K):
            m2 = jnp.max(s2, axis=0, keepdims=True)
            cand = jnp.where(s2 >= m2, iota, T)
            ik2 = jnp.min(cand, axis=0, keepdims=True)
            idx_ref[0, pl.ds(k, 1), :] = ik2 + b * T
            if k < K - 1:
                s2 = jnp.where(iota == ik2, -3.0, s2)


@functools.cache
def _make_topk_call(bc):
    return pl.pallas_call(
        _topk_body,
        grid=(bc, NT),
        in_specs=[pl.BlockSpec((1, C, T), lambda b, rt: (b, 0, 0)),
                  pl.BlockSpec((1, C, R), lambda b, rt: (b, 0, rt))],
        out_specs=[
            pl.BlockSpec((1, KPAD, R), lambda b, rt: (b, 0, rt)),
            pl.BlockSpec((1, R, C), lambda b, rt: (b, rt, 0)),
        ],
        out_shape=[
            jax.ShapeDtypeStruct((bc, KPAD, T), jnp.int32),
            jax.ShapeDtypeStruct((bc, T, C), jnp.float32),
        ],
    )


NBUF = 3


def _gather_body(nch, tab_ref, idx_ref, out_ref, idx_v,
                 rows0, rows1, rows2, sem0, sem1, sem2):
    per_w = nch * CH
    c = jax.lax.axis_index("c")
    s = jax.lax.axis_index("s")
    wid = s * NC + c
    base = wid * per_w
    rows = (rows0, rows1, rows2)
    sems = (sem0, sem1, sem2)
    pltpu.sync_copy(idx_ref.at[wid], idx_v)

    for u in range(NBUF):
        pltpu.async_copy(tab_ref.at[idx_v.at[u]], rows[u], sems[u])

    def chunk(i, carry):
        for u in range(NBUF):
            j = NBUF * i + u
            pltpu.make_async_copy(tab_ref.at[idx_v.at[j]],
                                  rows[u], sems[u]).wait()
            pltpu.sync_copy(rows[u], out_ref.at[pl.ds(base + j * CH, CH)])

            @pl.when(j + NBUF < nch)
            def _(u=u, j=j):
                pltpu.async_copy(tab_ref.at[idx_v.at[j + NBUF]],
                                 rows[u], sems[u])
        return carry

    jax.lax.fori_loop(0, nch // NBUF, chunk, 0)


@functools.cache
def _make_gather_call(bc):
    ntot = bc * T * K
    nch = ntot // (NW * CH)
    assert nch % NBUF == 0
    return pl.kernel(
        functools.partial(_gather_body, nch),
        out_type=jax.ShapeDtypeStruct((ntot, C), jnp.float32),
        mesh=plsc.VectorSubcoreMesh(core_axis_name="c", subcore_axis_name="s",
                                    num_cores=NC, num_subcores=NS),
        scratch_types=[
            pltpu.VMEM((nch, CH), jnp.int32),
            pltpu.VMEM((CH, C), jnp.float32),
            pltpu.VMEM((CH, C), jnp.float32),
            pltpu.VMEM((CH, C), jnp.float32),
            pltpu.SemaphoreType.DMA,
            pltpu.SemaphoreType.DMA,
            pltpu.SemaphoreType.DMA,
        ],
        compiler_params=pltpu.CompilerParams(use_tc_tiling_on_sc=False),
    )


def _conv_body(p_ref, w_ref, b_ref, o_ref):
    k = pl.program_id(1)
    contrib = jax.lax.dot_general(
        w_ref[0], p_ref[0], (((1,), (1,)), ((), ())),
        preferred_element_type=jnp.float32)

    @pl.when(k == 0)
    def _():
        o_ref[0] = contrib + b_ref[...]

    @pl.when(k != 0)
    def _():
        o_ref[0] = o_ref[0] + contrib


@functools.cache
def _make_conv_call(bc):
    return pl.pallas_call(
        _conv_body,
        grid=(bc, K),
        in_specs=[
            pl.BlockSpec((1, T, C), lambda b, k: (b * K + k, 0, 0)),
            pl.BlockSpec((1, C, C), lambda b, k: (k, 0, 0)),
            pl.BlockSpec((C, 1), lambda b, k: (0, 0)),
        ],
        out_specs=pl.BlockSpec((1, C, T), lambda b, k: (b, 0, 0)),
        out_shape=jax.ShapeDtypeStruct((bc, C, T), jnp.float32),
    )


def _chunk_pipeline(xf, w9, bias, bc):
    ntot = bc * T * K
    nch = ntot // (NW * CH)
    idx_g, xt = _make_topk_call(bc)(xf, xf)
    idx3 = idx_g[:, :K, :].reshape(NW, nch, CH)
    prime = _make_gather_call(bc)(xt.reshape(bc * T, C), idx3)
    p3 = prime.reshape(bc * K, T, C)
    return _make_conv_call(bc)(p3, w9, bias)


def kernel(x, W, b):
    xf = x.reshape(B, C, T)
    w9 = W.transpose(2, 0, 1)
    bias = b.reshape(C, 1)
    half = B // 2
    out0 = _chunk_pipeline(xf[:half], w9, bias, half)
    out1 = _chunk_pipeline(xf[half:], w9, bias, half)
    return jnp.concatenate([out0, out1], axis=0).reshape(B, C, 48, 48)


# speedup vs baseline: 2.0199x; 1.1369x over previous
import functools

import jax
import jax.numpy as jnp
from jax.experimental import pallas as pl
from jax.experimental.pallas import tpu as pltpu
from jax.experimental.pallas import tpu_sc as plsc

B = 8
C = 32
T = 48 * 48
K = 9
KPAD = 16

R = 768
NT = T // R

NC = 2
NS = 16
NW = NC * NS
CH = 96


def _topk_body(xf_ref, xr_ref, idx_ref, xt_ref):
    b = pl.program_id(0)
    xfb = xf_ref[0]
    norm = jnp.sqrt(jnp.sum(xfb * xfb, axis=0, keepdims=True))
    xn = xfb / jnp.maximum(norm, 1e-12)
    rows = xr_ref[0]
    norm_r = jnp.sqrt(jnp.sum(rows * rows, axis=0, keepdims=True))
    rows_n = rows / jnp.maximum(norm_r, 1e-12)
    xt_ref[0] = rows.T

    sim = jax.lax.dot_general(
        xn, rows_n, (((0,), (0,)), ((), ())),
        preferred_element_type=jnp.float32)
    sim = jnp.clip(sim, -1.0, 1.0)

    iota_i = jax.lax.broadcasted_iota(jnp.int32, (3, T), 1)
    row_id = jax.lax.broadcasted_iota(jnp.int32, (3, T), 0)
    lhs3 = jnp.where(
        row_id == 0, iota_i // 256,
        jnp.where(row_id == 1, iota_i % 256, 1)).astype(jnp.bfloat16)

    sim0 = sim
    m = jnp.max(sim, axis=0, keepdims=True)
    picks = []
    tied = False
    for k in range(K):
        ge = sim >= m
        gef = jnp.where(ge, 1.0, 0.0).astype(jnp.bfloat16)
        ms = jax.lax.dot_general(
            lhs3, gef, (((1,), (0,)), ((), ())),
            preferred_element_type=jnp.float32)
        tied = jnp.logical_or(tied, jnp.max(ms[2:3]) > 1.5)
        ikf = (ms[0:1] * 256.0 + ms[1:2]) / ms[2:3]
        ik = jnp.clip((ikf + 0.5).astype(jnp.int32), 0, T - 1)
        picks.append(ik)
        if k < K - 1:
            sim = jnp.where(ge, -3.0, sim)
            m = jnp.max(sim, axis=0, keepdims=True)
    idx = jnp.concatenate(picks + [jnp.zeros((KPAD - K, R), jnp.int32)], axis=0)
    idx_ref[0] = idx + b * T

    @pl.when(tied)
    def _():
        iota = jax.lax.broadcasted_iota(jnp.int32, (T, R), 0)
        s2 = sim0
        for k in range(K):
            m2 = jnp.max(s2, axis=0, keepdims=True)
            cand = jnp.where(s2 >= m2, iota, T)
            ik2 = jnp.min(cand, axis=0, keepdims=True)
            idx_ref[0, pl.ds(k, 1), :] = ik2 + b * T
            if k < K - 1:
                s2 = jnp.where(iota == ik2, -3.0, s2)


@functools.cache
def _make_topk_call(bc):
    return pl.pallas_call(
        _topk_body,
        grid=(bc, NT),
        in_specs=[pl.BlockSpec((1, C, T), lambda b, rt: (b, 0, 0)),
                  pl.BlockSpec((1, C, R), lambda b, rt: (b, 0, rt))],
        out_specs=[
            pl.BlockSpec((1, KPAD, R), lambda b, rt: (b, 0, rt)),
            pl.BlockSpec((1, R, C), lambda b, rt: (b, rt, 0)),
        ],
        out_shape=[
            jax.ShapeDtypeStruct((bc, KPAD, T), jnp.int32),
            jax.ShapeDtypeStruct((bc, T, C), jnp.float32),
        ],
    )


NBUF = 3


def _gather_body(nch, tab_ref, idx_ref, out_ref, idx_v,
                 rows0, rows1, rows2, sem0, sem1, sem2):
    per_w = nch * CH
    c = jax.lax.axis_index("c")
    s = jax.lax.axis_index("s")
    wid = s * NC + c
    base = wid * per_w
    rows = (rows0, rows1, rows2)
    sems = (sem0, sem1, sem2)
    pltpu.sync_copy(idx_ref.at[wid], idx_v)

    for u in range(NBUF):
        pltpu.async_copy(tab_ref.at[idx_v.at[u]], rows[u], sems[u])

    def chunk(i, carry):
        for u in range(NBUF):
            j = NBUF * i + u
            pltpu.make_async_copy(tab_ref.at[idx_v.at[j]],
                                  rows[u], sems[u]).wait()
            pltpu.sync_copy(rows[u], out_ref.at[pl.ds(base + j * CH, CH)])

            @pl.when(j + NBUF < nch)
            def _(u=u, j=j):
                pltpu.async_copy(tab_ref.at[idx_v.at[j + NBUF]],
                                 rows[u], sems[u])
        return carry

    jax.lax.fori_loop(0, nch // NBUF, chunk, 0)


@functools.cache
def _make_gather_call(bc):
    ntot = bc * T * K
    nch = ntot // (NW * CH)
    assert nch % NBUF == 0
    return pl.kernel(
        functools.partial(_gather_body, nch),
        out_type=jax.ShapeDtypeStruct((ntot, C), jnp.float32),
        mesh=plsc.VectorSubcoreMesh(core_axis_name="c", subcore_axis_name="s",
                                    num_cores=NC, num_subcores=NS),
        scratch_types=[
            pltpu.VMEM((nch, CH), jnp.int32),
            pltpu.VMEM((CH, C), jnp.float32),
            pltpu.VMEM((CH, C), jnp.float32),
            pltpu.VMEM((CH, C), jnp.float32),
            pltpu.SemaphoreType.DMA,
            pltpu.SemaphoreType.DMA,
            pltpu.SemaphoreType.DMA,
        ],
        compiler_params=pltpu.CompilerParams(use_tc_tiling_on_sc=False),
    )


def _conv_body(p_ref, w_ref, b_ref, o_ref):
    k = pl.program_id(1)
    contrib = jax.lax.dot_general(
        w_ref[0], p_ref[0], (((1,), (1,)), ((), ())),
        preferred_element_type=jnp.float32)

    @pl.when(k == 0)
    def _():
        o_ref[0] = contrib + b_ref[...]

    @pl.when(k != 0)
    def _():
        o_ref[0] = o_ref[0] + contrib


@functools.cache
def _make_conv_call(bc):
    return pl.pallas_call(
        _conv_body,
        grid=(bc, K),
        in_specs=[
            pl.BlockSpec((1, T, C), lambda b, k: (b * K + k, 0, 0)),
            pl.BlockSpec((1, C, C), lambda b, k: (k, 0, 0)),
            pl.BlockSpec((C, 1), lambda b, k: (0, 0)),
        ],
        out_specs=pl.BlockSpec((1, C, T), lambda b, k: (b, 0, 0)),
        out_shape=jax.ShapeDtypeStruct((bc, C, T), jnp.float32),
    )


def _chunk_pipeline(xf, w9, bias, bc):
    ntot = bc * T * K
    nch = ntot // (NW * CH)
    idx_g, xt = _make_topk_call(bc)(xf, xf)
    idx3 = idx_g[:, :K, :].reshape(NW, nch, CH)
    prime = _make_gather_call(bc)(xt.reshape(bc * T, C), idx3)
    p3 = prime.reshape(bc * K, T, C)
    return _make_conv_call(bc)(p3, w9, bias)


def kernel(x, W, b):
    xf = x.reshape(B, C, T)
    w9 = W.transpose(2, 0, 1)
    bias = b.reshape(C, 1)
    half = B // 2
    out0 = _chunk_pipeline(xf[:half], w9, bias, half)
    out1 = _chunk_pipeline(xf[half:], w9, bias, half)
    return jnp.concatenate([out0, out1], axis=0).reshape(B, C, 48, 48)
